# baseline (speedup 1.0000x reference)
import functools
import math

import jax
import jax.numpy as jnp
from jax.experimental import pallas as pl
from jax.experimental.pallas import tpu as pltpu
from jax.experimental.pallas import tpu_sc as plsc

B = 16
N = 32768
D = 256
H1 = 512
H2 = 256
C = 50

_NC = 2
_NS = 16
_NW = _NC * _NS
_RPW = N // _NW
_CH = 128
_NCH = _RPW // _CH
_DL = D // 16


def _sc_segsum_body(feat_hbm, offs_hbm, out_hbm,
                    offs_v, buf0, buf1, acc, sem0, sem1):
    cid = jax.lax.axis_index("c")
    sid = jax.lax.axis_index("s")
    wid = sid * _NC + cid
    lo = wid * _RPW

    pltpu.sync_copy(offs_hbm, offs_v)
    offv_lo = offs_v[pl.ds(0, 16)]
    offv_hi = offs_v[pl.ds(16, 16)]
    off_sc = [offv_lo[s] for s in range(16)] + [offv_hi[0]]

    zero = jnp.zeros((16,), jnp.float32)
    for s in range(B):
        for dd in range(_DL):
            acc[s, pl.ds(dd * 16, 16)] = zero

    pltpu.async_copy(feat_hbm.at[pl.ds(lo, _CH)], buf0, sem0)
    pltpu.async_copy(feat_hbm.at[pl.ds(lo + _CH, _CH)], buf1, sem1)

    def process(c, buf):
        chunk_lo = lo + c * _CH
        for s in range(B):
            a = jnp.maximum(off_sc[s], chunk_lo)
            b = jnp.minimum(off_sc[s + 1], chunk_lo + _CH)

            def body(r, carry):
                rl = r - chunk_lo
                return tuple(carry[dd] + buf[rl, pl.ds(dd * 16, 16)]
                             for dd in range(_DL))

            res = jax.lax.fori_loop(a, b, body, tuple(zero for _ in range(_DL)))
            for dd in range(_DL):
                plsc.addupdate(acc.at[s, pl.ds(dd * 16, 16)], res[dd])

    def loop_body(j, carry):
        c0 = j * 2
        for half, (buf, sem) in enumerate(((buf0, sem0), (buf1, sem1))):
            c = c0 + half
            pltpu.make_async_copy(feat_hbm.at[pl.ds(0, _CH)], buf, sem).wait()
            process(c, buf)

            @pl.when(c + 2 < _NCH)
            def _():
                pltpu.async_copy(
                    feat_hbm.at[pl.ds(lo + (c + 2) * _CH, _CH)], buf, sem)
        return carry

    jax.lax.fori_loop(0, _NCH // 2, loop_body, 0)
    pltpu.sync_copy(acc, out_hbm.at[wid])


_sc_segsum = functools.partial(
    pl.kernel,
    out_type=jax.ShapeDtypeStruct((_NW, B, D), jnp.float32),
    mesh=plsc.VectorSubcoreMesh(core_axis_name="c", subcore_axis_name="s"),
    scratch_types=[
        pltpu.VMEM((32,), jnp.int32),
        pltpu.VMEM((_CH, D), jnp.float32),
        pltpu.VMEM((_CH, D), jnp.float32),
        pltpu.VMEM((B, D), jnp.float32),
        pltpu.SemaphoreType.DMA,
        pltpu.SemaphoreType.DMA,
    ],
)(_sc_segsum_body)


def _erf(x):
    a1, a2, a3, a4, a5 = (0.254829592, -0.284496736, 1.421413741,
                          -1.453152027, 1.061405429)
    p = 0.3275911
    ax = jnp.abs(x)
    t = 1.0 / (1.0 + p * ax)
    poly = t * (a1 + t * (a2 + t * (a3 + t * (a4 + t * a5))))
    y = 1.0 - poly * jnp.exp(-ax * ax)
    return jnp.sign(x) * y


def _gelu(x):
    return 0.5 * x * (1.0 + _erf(x * (1.0 / math.sqrt(2.0))))


def _layernorm(x, g, b, eps=1e-5):
    m = jnp.mean(x, axis=-1, keepdims=True)
    v = jnp.mean((x - m) ** 2, axis=-1, keepdims=True)
    return (x - m) * jax.lax.rsqrt(v + eps) * g + b


def _mlp_body(partials_ref, counts_ref, W1_ref, b1_ref, g1_ref, be1_ref,
              W2_ref, b2_ref, g2_ref, be2_ref, W3_ref, b3_ref, out_ref):
    sums = jnp.sum(partials_ref[...], axis=0)
    means = sums / jnp.maximum(counts_ref[...], 1.0)
    h = jnp.dot(means, W1_ref[...], preferred_element_type=jnp.float32)
    h = h + b1_ref[...]
    h = _layernorm(h, g1_ref[...], be1_ref[...])
    h = _gelu(h)
    h = jnp.dot(h, W2_ref[...], preferred_element_type=jnp.float32)
    h = h + b2_ref[...]
    h = _layernorm(h, g2_ref[...], be2_ref[...])
    h = _gelu(h)
    out = jnp.dot(h, W3_ref[...], preferred_element_type=jnp.float32)
    out_ref[...] = out + b3_ref[...]


@jax.jit
def kernel(feat, offsets, W1, b1, g1, be1, W2, b2, g2, be2, W3, b3):
    off = offsets.astype(jnp.int32)
    counts = (off[1:] - off[:-1]).reshape(B, 1).astype(jnp.float32)
    offs_pad = jnp.concatenate(
        [off, jnp.full((32 - (B + 1),), N, dtype=jnp.int32)])

    partials = _sc_segsum(feat, offs_pad)

    out = pl.pallas_call(
        _mlp_body,
        in_specs=[
            pl.BlockSpec((_NW, B, D), lambda: (0, 0, 0)),
            pl.BlockSpec((B, 1), lambda: (0, 0)),
            pl.BlockSpec((D, H1), lambda: (0, 0)),
            pl.BlockSpec((1, H1), lambda: (0, 0)),
            pl.BlockSpec((1, H1), lambda: (0, 0)),
            pl.BlockSpec((1, H1), lambda: (0, 0)),
            pl.BlockSpec((H1, H2), lambda: (0, 0)),
            pl.BlockSpec((1, H2), lambda: (0, 0)),
            pl.BlockSpec((1, H2), lambda: (0, 0)),
            pl.BlockSpec((1, H2), lambda: (0, 0)),
            pl.BlockSpec((H2, C), lambda: (0, 0)),
            pl.BlockSpec((1, C), lambda: (0, 0)),
        ],
        out_specs=pl.BlockSpec((B, C), lambda: (0, 0)),
        out_shape=jax.ShapeDtypeStruct((B, C), jnp.float32),
    )(partials, counts, W1, b1.reshape(1, H1), g1.reshape(1, H1),
      be1.reshape(1, H1), W2, b2.reshape(1, H2), g2.reshape(1, H2),
      be2.reshape(1, H2), W3, b3.reshape(1, C))
    return out
